# baseline (speedup 1.0000x reference)
import functools

import jax
import jax.numpy as jnp
from jax import lax
from jax.experimental import pallas as pl
from jax.experimental.pallas import tpu as pltpu
from jax.experimental.pallas import tpu_sc as plsc

N = 10000
NP = 10240
E = 160000
E_PAD = 163840
D = 256
HD = 128
NSUB = 16
CH = 128
NBLK = 10
BCH = 8
NCH = NBLK * BCH
RPT = NP // NSUB
ZR = 64
RBLK = 512


def _unpack_group(ib, sbuf, dbuf, unpack_src):
  half = BCH // 2
  for k in range(BCH):
    for g in range(4):
      w0 = (CH // 2) * k + 16 * g
      r, col = divmod(w0, CH)
      if unpack_src:
        w_s = ib[r, pl.ds(col, 16)]
        sbuf[k, pl.ds(32 * g, 16)] = w_s & 0xFFFF
        sbuf[k, pl.ds(32 * g + 16, 16)] = jnp.right_shift(w_s, 16)
      w_d = ib[half + r, pl.ds(col, 16)]
      dbuf[k, pl.ds(32 * g, 16)] = w_d & 0xFFFF
      dbuf[k, pl.ds(32 * g + 16, 16)] = jnp.right_shift(w_d, 16)


def _segmean_body(xL, xR, idxp, zer,
                  outL, outR,
                  ib, sbuf, dbuf, rows0, rows1, agg_sh, sem0, sem1):
  c = lax.axis_index("c")
  s = lax.axis_index("s")

  def pipe(x_hbm, out_hbm):
    def zinit(i, carry):
      pltpu.sync_copy(zer, agg_sh.at[pl.ds(s * RPT + i * ZR, ZR)])
      return carry
    lax.fori_loop(0, RPT // ZR, zinit, 0)
    plsc.subcore_barrier()

    def load_group(g):
      pltpu.sync_copy(idxp.at[s, g], ib)
      _unpack_group(ib, sbuf, dbuf, True)

    def gather(k, rows, sem):
      return pltpu.make_async_copy(x_hbm.at[sbuf.at[k]], rows, sem)

    load_group(0)
    gather(0, rows0, sem0).start()

    def grp(g, carry):
      bufs = ((rows0, sem0), (rows1, sem1))
      for k in range(BCH):
        rows, sem = bufs[k % 2]
        gather(k, rows, sem).wait()
        if k + 1 < BCH:
          nrows, nsem = bufs[(k + 1) % 2]
          gather(k + 1, nrows, nsem).start()
        pltpu.sync_copy(rows, agg_sh.at[dbuf.at[k]], add=True)
      load_group(lax.rem(g + 1, NBLK))
      gather(0, rows0, sem0).start()
      return carry

    lax.fori_loop(0, NBLK, grp, 0)
    gather(0, rows0, sem0).wait()
    plsc.subcore_barrier()

    pltpu.sync_copy(agg_sh.at[pl.ds(s * RPT, RPT)], out_hbm.at[pl.ds(s * RPT, RPT)])

  @pl.when(c == 0)
  def _():
    pipe(xL, outL)

  @pl.when(c == 1)
  def _():
    pipe(xR, outR)


def _make_segmean():
  mesh = plsc.VectorSubcoreMesh(core_axis_name="c", subcore_axis_name="s")
  out_type = [jax.ShapeDtypeStruct((NP, HD), jnp.float32),
              jax.ShapeDtypeStruct((NP, HD), jnp.float32)]
  scratch = [
      pltpu.VMEM((BCH, CH), jnp.int32),
      pltpu.VMEM((BCH, CH), jnp.int32),
      pltpu.VMEM((BCH, CH), jnp.int32),
      pltpu.VMEM((CH, HD), jnp.float32),
      pltpu.VMEM((CH, HD), jnp.float32),
      pltpu.VMEM_SHARED((NP, HD), jnp.float32),
      pltpu.SemaphoreType.DMA,
      pltpu.SemaphoreType.DMA,
  ]
  return pl.kernel(_segmean_body, out_type=out_type, mesh=mesh,
                   scratch_types=scratch)


def _degree_body(idxp, zer, ones_h,
                 out0, out1,
                 ib, dbuf, ones_v, deg_sh):
  c = lax.axis_index("c")
  s = lax.axis_index("s")

  def zinit(i, carry):
    pltpu.sync_copy(zer, deg_sh.at[pl.ds(s * RPT + i * ZR, ZR)])
    return carry
  lax.fori_loop(0, RPT // ZR, zinit, 0)
  pltpu.sync_copy(ones_h, ones_v)
  plsc.subcore_barrier()

  def grp(b, carry):
    pltpu.sync_copy(idxp.at[s, c * (NBLK // 2) + b], ib)
    _unpack_group(ib, None, dbuf, False)
    for k in range(BCH):
      pltpu.sync_copy(ones_v, deg_sh.at[dbuf.at[k]], add=True)
    return carry

  lax.fori_loop(0, NBLK // 2, grp, 0)
  plsc.subcore_barrier()

  @pl.when(c == 0)
  def _():
    pltpu.sync_copy(deg_sh.at[pl.ds(s * RPT, RPT)], out0.at[pl.ds(s * RPT, RPT)])

  @pl.when(c == 1)
  def _():
    pltpu.sync_copy(deg_sh.at[pl.ds(s * RPT, RPT)], out1.at[pl.ds(s * RPT, RPT)])


def _make_degree():
  mesh = plsc.VectorSubcoreMesh(core_axis_name="c", subcore_axis_name="s")
  out_type = [jax.ShapeDtypeStruct((NP, HD), jnp.float32),
              jax.ShapeDtypeStruct((NP, HD), jnp.float32)]
  scratch = [
      pltpu.VMEM((BCH, CH), jnp.int32),
      pltpu.VMEM((BCH, CH), jnp.int32),
      pltpu.VMEM((CH, HD), jnp.float32),
      pltpu.VMEM_SHARED((NP, HD), jnp.float32),
  ]
  return pl.kernel(_degree_body, out_type=out_type, mesh=mesh,
                   scratch_types=scratch)


def _layer_body(relu, aL_ref, aR_ref, d0_ref, d1_ref, xL_ref, xR_ref,
                wl_ref, bl_ref, wr_ref, oL_ref, oR_ref):
  s = 1.0 / jnp.maximum(d0_ref[:, 0:1] + d1_ref[:, 0:1], 1.0)
  aL = aL_ref[...] * s
  aR = aR_ref[...] * s
  wl = wl_ref[...]
  wr = wr_ref[...]
  o = (jnp.dot(aL, wl[:HD], preferred_element_type=jnp.float32)
       + jnp.dot(aR, wl[HD:], preferred_element_type=jnp.float32)
       + jnp.dot(xL_ref[...], wr[:HD], preferred_element_type=jnp.float32)
       + jnp.dot(xR_ref[...], wr[HD:], preferred_element_type=jnp.float32)
       + bl_ref[...][None, :])
  if relu:
    o = jnp.maximum(o, 0.0)
  oL_ref[...] = o[:, :HD]
  oR_ref[...] = o[:, HD:]


def _tc_layer(aL, aR, d0, d1, xL, xR, Wl, bl, Wr, relu):
  grid = (NP // RBLK,)
  half = pl.BlockSpec((RBLK, HD), lambda i: (i, 0))
  return pl.pallas_call(
      functools.partial(_layer_body, relu),
      grid=grid,
      in_specs=[half, half, half, half, half, half,
                pl.BlockSpec((D, D), lambda i: (0, 0)),
                pl.BlockSpec((D,), lambda i: (0,)),
                pl.BlockSpec((D, D), lambda i: (0, 0))],
      out_specs=[half, half],
      out_shape=[jax.ShapeDtypeStruct((NP, HD), jnp.float32),
                 jax.ShapeDtypeStruct((NP, HD), jnp.float32)],
  )(aL, aR, d0, d1, xL, xR, Wl, bl, Wr)


def _head_body(aL_ref, aR_ref, d0_ref, d1_ref, xL_ref, xR_ref, wl_ref, bl_ref,
               wr_ref, wp1_ref, bp1_ref, wp2_ref, bp2_ref, wn1_ref, bn1_ref,
               wn2_ref, bn2_ref, z_ref, nul_ref):
  s = 1.0 / jnp.maximum(d0_ref[:, 0:1] + d1_ref[:, 0:1], 1.0)
  aL = aL_ref[...] * s
  aR = aR_ref[...] * s
  wl = wl_ref[...]
  wr = wr_ref[...]
  h = (jnp.dot(aL, wl[:HD], preferred_element_type=jnp.float32)
       + jnp.dot(aR, wl[HD:], preferred_element_type=jnp.float32)
       + jnp.dot(xL_ref[...], wr[:HD], preferred_element_type=jnp.float32)
       + jnp.dot(xR_ref[...], wr[HD:], preferred_element_type=jnp.float32)
       + bl_ref[...][None, :])
  t = jnp.maximum(jnp.dot(h, wp1_ref[...], preferred_element_type=jnp.float32)
                  + bp1_ref[...][None, :], 0.0)
  z0 = jnp.dot(t, wp2_ref[...], preferred_element_type=jnp.float32) + bp2_ref[...][None, :]
  nrm = jnp.sqrt(jnp.sum(z0 * z0, axis=1, keepdims=True))
  z = z0 / jnp.maximum(nrm, 1e-12)
  wn1 = wn1_ref[...]
  y = jnp.maximum(jnp.dot(h, wn1[:D], preferred_element_type=jnp.float32)
                  + jnp.dot(z, wn1[D:], preferred_element_type=jnp.float32)
                  + bn1_ref[...][None, :], 0.0)
  nul = jnp.sum(y * wn2_ref[...][None, :], axis=1) + bn2_ref[...]
  z_ref[...] = z
  nul_ref[...] = nul


def _tc_layer2_heads(aL, aR, d0, d1, xL, xR, Wl, bl, Wr,
                     Wp1, bp1, Wp2, bp2, Wn1, bn1, wn2, bn2):
  grid = (NP // RBLK,)
  half = pl.BlockSpec((RBLK, HD), lambda i: (i, 0))
  full = lambda shape: pl.BlockSpec(shape, (lambda i: (0,) * len(shape)))
  return pl.pallas_call(
      _head_body,
      grid=grid,
      in_specs=[half, half, half, half, half, half,
                full((D, D)), full((D,)), full((D, D)),
                full((D, HD)), full((HD,)), full((HD, HD)), full((HD,)),
                full((D + HD, 64)), full((64,)), full((64,)), full((1,))],
      out_specs=[pl.BlockSpec((RBLK, HD), lambda i: (i, 0)),
                 pl.BlockSpec((RBLK,), lambda i: (i,))],
      out_shape=[jax.ShapeDtypeStruct((NP, HD), jnp.float32),
                 jax.ShapeDtypeStruct((NP,), jnp.float32)],
  )(aL, aR, d0, d1, xL, xR, Wl, bl, Wr, Wp1, bp1, Wp2, bp2, Wn1, bn1, wn2, bn2)


def kernel(A_x, B_x, A_edge_index, B_edge_index,
           Wl0, bl0, Wr0, Wl1, bl1, Wr1, Wl2, bl2, Wr2,
           Wp1, bp1, Wp2, bp2, Wn1, bn1, Wn2, bn2):
  segmean = _make_segmean()
  degree = _make_degree()

  zeros_h = jnp.zeros((ZR, HD), jnp.float32)
  ones_h = jnp.ones((CH, HD), jnp.float32)
  wn2 = Wn2[:, 0]

  pad = E_PAD - E

  def prep_edges(ei):
    src = ei[0].astype(jnp.int32)
    dst = ei[1].astype(jnp.int32)
    srcp = jnp.concatenate([src, jnp.zeros((pad,), jnp.int32)])
    dpad = N + (jnp.arange(pad, dtype=jnp.int32) % (NP - N))
    dstp = jnp.concatenate([dst, dpad])

    def pack(a):
      a2 = a.reshape(NSUB, NBLK, BCH, CH // 2, 2)
      w = a2[..., 0] | (a2[..., 1] << 16)
      return w.reshape(NSUB, NBLK, BCH // 2, CH)

    return jnp.concatenate([pack(srcp), pack(dstp)], axis=2)

  def prep_x(x):
    xp = jnp.zeros((NP, D), jnp.float32).at[:N].set(x)
    return xp[:, :HD], xp[:, HD:]

  def branch(x, ei):
    idx16 = prep_edges(ei)
    xL, xR = prep_x(x)
    d0, d1 = degree(idx16, zeros_h, ones_h)
    aL0, aR0 = segmean(xL, xR, idx16, zeros_h)
    h1L, h1R = _tc_layer(aL0, aR0, d0, d1, xL, xR, Wl0, bl0, Wr0, True)
    aL1, aR1 = segmean(h1L, h1R, idx16, zeros_h)
    h2L, h2R = _tc_layer(aL1, aR1, d0, d1, h1L, h1R, Wl1, bl1, Wr1, True)
    aL2, aR2 = segmean(h2L, h2R, idx16, zeros_h)
    z, nul = _tc_layer2_heads(aL2, aR2, d0, d1, h2L, h2R, Wl2, bl2, Wr2,
                              Wp1, bp1, Wp2, bp2, Wn1, bn1, wn2, bn2)
    return z[:N], nul[:N]

  zA, nulA = branch(A_x, A_edge_index)
  zB, nulB = branch(B_x, B_edge_index)
  return (zA, zB, nulA, nulB)


# speedup vs baseline: 1.0005x; 1.0005x over previous
import functools

import jax
import jax.numpy as jnp
from jax import lax
from jax.experimental import pallas as pl
from jax.experimental.pallas import tpu as pltpu
from jax.experimental.pallas import tpu_sc as plsc

N = 10000
NP = 10240
E = 160000
E_PAD = 163840
D = 256
HD = 128
NSUB = 16
CH = 128
NBLK = 10
BCH = 8
NCH = NBLK * BCH
RPT = NP // NSUB
ZR = 64
RBLK = 512


def _unpack_group(ib, sbuf, dbuf, unpack_src):
  half = BCH // 2
  for k in range(BCH):
    for g in range(4):
      w0 = (CH // 2) * k + 16 * g
      r, col = divmod(w0, CH)
      if unpack_src:
        w_s = ib[r, pl.ds(col, 16)]
        sbuf[k, pl.ds(32 * g, 16)] = w_s & 0xFFFF
        sbuf[k, pl.ds(32 * g + 16, 16)] = jnp.right_shift(w_s, 16)
      w_d = ib[half + r, pl.ds(col, 16)]
      dbuf[k, pl.ds(32 * g, 16)] = w_d & 0xFFFF
      dbuf[k, pl.ds(32 * g + 16, 16)] = jnp.right_shift(w_d, 16)


def _segmean_body(xL, xR, idxp, zer,
                  outL, outR,
                  ib, sbuf, dbuf, rows0, rows1, agg_sh,
                  gsem0, gsem1, ssem0, ssem1):
  c = lax.axis_index("c")
  s = lax.axis_index("s")

  def pipe(x_hbm, out_hbm):
    def zinit(i, carry):
      pltpu.sync_copy(zer, agg_sh.at[pl.ds(s * RPT + i * ZR, ZR)])
      return carry
    lax.fori_loop(0, RPT // ZR, zinit, 0)
    plsc.subcore_barrier()

    def load_group(g):
      pltpu.sync_copy(idxp.at[s, g], ib)
      _unpack_group(ib, sbuf, dbuf, True)

    def gather(k, rows, sem):
      return pltpu.make_async_copy(x_hbm.at[sbuf.at[k]], rows, sem)

    def scatter(k, rows, sem):
      return pltpu.async_copy(rows, agg_sh.at[dbuf.at[k]], sem, add=True)

    load_group(0)
    gather(0, rows0, gsem0).start()

    def grp(g, carry):
      bufs = ((rows0, gsem0, ssem0), (rows1, gsem1, ssem1))
      for k in range(BCH):
        rows, gsem, ssem = bufs[k % 2]
        gather(k, rows, gsem).wait()
        sc = scatter(k, rows, ssem)
        if k + 1 < BCH:
          nrows, ngsem, nssem = bufs[(k + 1) % 2]
          if k >= 1:
            pltpu.make_async_copy(nrows, agg_sh.at[dbuf.at[k - 1]],
                                  nssem).wait()
          gather(k + 1, nrows, ngsem).start()
      pltpu.make_async_copy(rows0, agg_sh.at[dbuf.at[BCH - 2]], ssem0).wait()
      pltpu.make_async_copy(rows1, agg_sh.at[dbuf.at[BCH - 1]], ssem1).wait()
      load_group(lax.rem(g + 1, NBLK))
      gather(0, rows0, gsem0).start()
      return carry

    lax.fori_loop(0, NBLK, grp, 0)
    gather(0, rows0, gsem0).wait()
    plsc.subcore_barrier()

    pltpu.sync_copy(agg_sh.at[pl.ds(s * RPT, RPT)], out_hbm.at[pl.ds(s * RPT, RPT)])

  @pl.when(c == 0)
  def _():
    pipe(xL, outL)

  @pl.when(c == 1)
  def _():
    pipe(xR, outR)


def _make_segmean():
  mesh = plsc.VectorSubcoreMesh(core_axis_name="c", subcore_axis_name="s")
  out_type = [jax.ShapeDtypeStruct((NP, HD), jnp.float32),
              jax.ShapeDtypeStruct((NP, HD), jnp.float32)]
  scratch = [
      pltpu.VMEM((BCH, CH), jnp.int32),
      pltpu.VMEM((BCH, CH), jnp.int32),
      pltpu.VMEM((BCH, CH), jnp.int32),
      pltpu.VMEM((CH, HD), jnp.float32),
      pltpu.VMEM((CH, HD), jnp.float32),
      pltpu.VMEM_SHARED((NP, HD), jnp.float32),
      pltpu.SemaphoreType.DMA,
      pltpu.SemaphoreType.DMA,
      pltpu.SemaphoreType.DMA,
      pltpu.SemaphoreType.DMA,
  ]
  return pl.kernel(_segmean_body, out_type=out_type, mesh=mesh,
                   scratch_types=scratch)


def _degree_body(idxp, zer, ones_h,
                 out0, out1,
                 ib, dbuf, ones_v, deg_sh):
  c = lax.axis_index("c")
  s = lax.axis_index("s")

  def zinit(i, carry):
    pltpu.sync_copy(zer, deg_sh.at[pl.ds(s * RPT + i * ZR, ZR)])
    return carry
  lax.fori_loop(0, RPT // ZR, zinit, 0)
  pltpu.sync_copy(ones_h, ones_v)
  plsc.subcore_barrier()

  def grp(b, carry):
    pltpu.sync_copy(idxp.at[s, c * (NBLK // 2) + b], ib)
    _unpack_group(ib, None, dbuf, False)
    for k in range(BCH):
      pltpu.sync_copy(ones_v, deg_sh.at[dbuf.at[k]], add=True)
    return carry

  lax.fori_loop(0, NBLK // 2, grp, 0)
  plsc.subcore_barrier()

  @pl.when(c == 0)
  def _():
    pltpu.sync_copy(deg_sh.at[pl.ds(s * RPT, RPT)], out0.at[pl.ds(s * RPT, RPT)])

  @pl.when(c == 1)
  def _():
    pltpu.sync_copy(deg_sh.at[pl.ds(s * RPT, RPT)], out1.at[pl.ds(s * RPT, RPT)])


def _make_degree():
  mesh = plsc.VectorSubcoreMesh(core_axis_name="c", subcore_axis_name="s")
  out_type = [jax.ShapeDtypeStruct((NP, HD), jnp.float32),
              jax.ShapeDtypeStruct((NP, HD), jnp.float32)]
  scratch = [
      pltpu.VMEM((BCH, CH), jnp.int32),
      pltpu.VMEM((BCH, CH), jnp.int32),
      pltpu.VMEM((CH, HD), jnp.float32),
      pltpu.VMEM_SHARED((NP, HD), jnp.float32),
  ]
  return pl.kernel(_degree_body, out_type=out_type, mesh=mesh,
                   scratch_types=scratch)


def _layer_body(relu, aL_ref, aR_ref, d0_ref, d1_ref, xL_ref, xR_ref,
                wl_ref, bl_ref, wr_ref, oL_ref, oR_ref):
  s = 1.0 / jnp.maximum(d0_ref[:, 0:1] + d1_ref[:, 0:1], 1.0)
  aL = aL_ref[...] * s
  aR = aR_ref[...] * s
  wl = wl_ref[...]
  wr = wr_ref[...]
  o = (jnp.dot(aL, wl[:HD], preferred_element_type=jnp.float32)
       + jnp.dot(aR, wl[HD:], preferred_element_type=jnp.float32)
       + jnp.dot(xL_ref[...], wr[:HD], preferred_element_type=jnp.float32)
       + jnp.dot(xR_ref[...], wr[HD:], preferred_element_type=jnp.float32)
       + bl_ref[...][None, :])
  if relu:
    o = jnp.maximum(o, 0.0)
  oL_ref[...] = o[:, :HD]
  oR_ref[...] = o[:, HD:]


def _tc_layer(aL, aR, d0, d1, xL, xR, Wl, bl, Wr, relu):
  grid = (NP // RBLK,)
  half = pl.BlockSpec((RBLK, HD), lambda i: (i, 0))
  return pl.pallas_call(
      functools.partial(_layer_body, relu),
      grid=grid,
      in_specs=[half, half, half, half, half, half,
                pl.BlockSpec((D, D), lambda i: (0, 0)),
                pl.BlockSpec((D,), lambda i: (0,)),
                pl.BlockSpec((D, D), lambda i: (0, 0))],
      out_specs=[half, half],
      out_shape=[jax.ShapeDtypeStruct((NP, HD), jnp.float32),
                 jax.ShapeDtypeStruct((NP, HD), jnp.float32)],
  )(aL, aR, d0, d1, xL, xR, Wl, bl, Wr)


def _head_body(aL_ref, aR_ref, d0_ref, d1_ref, xL_ref, xR_ref, wl_ref, bl_ref,
               wr_ref, wp1_ref, bp1_ref, wp2_ref, bp2_ref, wn1_ref, bn1_ref,
               wn2_ref, bn2_ref, z_ref, nul_ref):
  s = 1.0 / jnp.maximum(d0_ref[:, 0:1] + d1_ref[:, 0:1], 1.0)
  aL = aL_ref[...] * s
  aR = aR_ref[...] * s
  wl = wl_ref[...]
  wr = wr_ref[...]
  h = (jnp.dot(aL, wl[:HD], preferred_element_type=jnp.float32)
       + jnp.dot(aR, wl[HD:], preferred_element_type=jnp.float32)
       + jnp.dot(xL_ref[...], wr[:HD], preferred_element_type=jnp.float32)
       + jnp.dot(xR_ref[...], wr[HD:], preferred_element_type=jnp.float32)
       + bl_ref[...][None, :])
  t = jnp.maximum(jnp.dot(h, wp1_ref[...], preferred_element_type=jnp.float32)
                  + bp1_ref[...][None, :], 0.0)
  z0 = jnp.dot(t, wp2_ref[...], preferred_element_type=jnp.float32) + bp2_ref[...][None, :]
  nrm = jnp.sqrt(jnp.sum(z0 * z0, axis=1, keepdims=True))
  z = z0 / jnp.maximum(nrm, 1e-12)
  wn1 = wn1_ref[...]
  y = jnp.maximum(jnp.dot(h, wn1[:D], preferred_element_type=jnp.float32)
                  + jnp.dot(z, wn1[D:], preferred_element_type=jnp.float32)
                  + bn1_ref[...][None, :], 0.0)
  nul = jnp.sum(y * wn2_ref[...][None, :], axis=1) + bn2_ref[...]
  z_ref[...] = z
  nul_ref[...] = nul


def _tc_layer2_heads(aL, aR, d0, d1, xL, xR, Wl, bl, Wr,
                     Wp1, bp1, Wp2, bp2, Wn1, bn1, wn2, bn2):
  grid = (NP // RBLK,)
  half = pl.BlockSpec((RBLK, HD), lambda i: (i, 0))
  full = lambda shape: pl.BlockSpec(shape, (lambda i: (0,) * len(shape)))
  return pl.pallas_call(
      _head_body,
      grid=grid,
      in_specs=[half, half, half, half, half, half,
                full((D, D)), full((D,)), full((D, D)),
                full((D, HD)), full((HD,)), full((HD, HD)), full((HD,)),
                full((D + HD, 64)), full((64,)), full((64,)), full((1,))],
      out_specs=[pl.BlockSpec((RBLK, HD), lambda i: (i, 0)),
                 pl.BlockSpec((RBLK,), lambda i: (i,))],
      out_shape=[jax.ShapeDtypeStruct((NP, HD), jnp.float32),
                 jax.ShapeDtypeStruct((NP,), jnp.float32)],
  )(aL, aR, d0, d1, xL, xR, Wl, bl, Wr, Wp1, bp1, Wp2, bp2, Wn1, bn1, wn2, bn2)


def kernel(A_x, B_x, A_edge_index, B_edge_index,
           Wl0, bl0, Wr0, Wl1, bl1, Wr1, Wl2, bl2, Wr2,
           Wp1, bp1, Wp2, bp2, Wn1, bn1, Wn2, bn2):
  segmean = _make_segmean()
  degree = _make_degree()

  zeros_h = jnp.zeros((ZR, HD), jnp.float32)
  ones_h = jnp.ones((CH, HD), jnp.float32)
  wn2 = Wn2[:, 0]

  pad = E_PAD - E

  def prep_edges(ei):
    src = ei[0].astype(jnp.int32)
    dst = ei[1].astype(jnp.int32)
    srcp = jnp.concatenate([src, jnp.zeros((pad,), jnp.int32)])
    dpad = N + (jnp.arange(pad, dtype=jnp.int32) % (NP - N))
    dstp = jnp.concatenate([dst, dpad])

    def pack(a):
      a2 = a.reshape(NSUB, NBLK, BCH, CH // 2, 2)
      w = a2[..., 0] | (a2[..., 1] << 16)
      return w.reshape(NSUB, NBLK, BCH // 2, CH)

    return jnp.concatenate([pack(srcp), pack(dstp)], axis=2)

  def prep_x(x):
    xp = jnp.zeros((NP, D), jnp.float32).at[:N].set(x)
    return xp[:, :HD], xp[:, HD:]

  def branch(x, ei):
    idx16 = prep_edges(ei)
    xL, xR = prep_x(x)
    d0, d1 = degree(idx16, zeros_h, ones_h)
    aL0, aR0 = segmean(xL, xR, idx16, zeros_h)
    h1L, h1R = _tc_layer(aL0, aR0, d0, d1, xL, xR, Wl0, bl0, Wr0, True)
    aL1, aR1 = segmean(h1L, h1R, idx16, zeros_h)
    h2L, h2R = _tc_layer(aL1, aR1, d0, d1, h1L, h1R, Wl1, bl1, Wr1, True)
    aL2, aR2 = segmean(h2L, h2R, idx16, zeros_h)
    z, nul = _tc_layer2_heads(aL2, aR2, d0, d1, h2L, h2R, Wl2, bl2, Wr2,
                              Wp1, bp1, Wp2, bp2, Wn1, bn1, wn2, bn2)
    return z[:N], nul[:N]

  zA, nulA = branch(A_x, A_edge_index)
  zB, nulB = branch(B_x, B_edge_index)
  return (zA, zB, nulA, nulB)


# speedup vs baseline: 1.0042x; 1.0037x over previous
import functools

import jax
import jax.numpy as jnp
from jax import lax
from jax.experimental import pallas as pl
from jax.experimental.pallas import tpu as pltpu
from jax.experimental.pallas import tpu_sc as plsc

N = 10000
NP = 10240
E = 160000
E_PAD = 163840
D = 256
HD = 128
NSUB = 16
CH = 128
NBLK = 10
BCH = 8
NCH = NBLK * BCH
RPT = NP // NSUB
ZR = 64
RBLK = 512


def _unpack_group(ib, sbuf, dbuf):
  half = BCH // 2
  for k in range(BCH):
    for g in range(4):
      w0 = (CH // 2) * k + 16 * g
      r, col = divmod(w0, CH)
      w_s = ib[r, pl.ds(col, 16)]
      sbuf[k, pl.ds(32 * g, 16)] = w_s & 0xFFFF
      sbuf[k, pl.ds(32 * g + 16, 16)] = jnp.right_shift(w_s, 16)
      w_d = ib[half + r, pl.ds(col, 16)]
      dbuf[k, pl.ds(32 * g, 16)] = w_d & 0xFFFF
      dbuf[k, pl.ds(32 * g + 16, 16)] = jnp.right_shift(w_d, 16)


def _segmean_body(xL, xR, idxp, zer,
                  outL, outR,
                  ib, sbuf, dbuf, rows0, rows1, agg_sh,
                  gsem0, gsem1, ssem0, ssem1):
  c = lax.axis_index("c")
  s = lax.axis_index("s")

  def pipe(x_hbm, out_hbm):
    def zinit(i, carry):
      pltpu.sync_copy(zer, agg_sh.at[pl.ds(s * RPT + i * ZR, ZR)])
      return carry
    lax.fori_loop(0, RPT // ZR, zinit, 0)
    plsc.subcore_barrier()

    def load_group(g):
      pltpu.sync_copy(idxp.at[s, g], ib)
      _unpack_group(ib, sbuf, dbuf)

    def gather(k, rows, sem):
      return pltpu.make_async_copy(x_hbm.at[sbuf.at[k]], rows, sem)

    def scatter(k, rows, sem):
      return pltpu.async_copy(rows, agg_sh.at[dbuf.at[k]], sem, add=True)

    load_group(0)
    gather(0, rows0, gsem0).start()

    def grp(g, carry):
      bufs = ((rows0, gsem0, ssem0), (rows1, gsem1, ssem1))
      for k in range(BCH):
        rows, gsem, ssem = bufs[k % 2]
        gather(k, rows, gsem).wait()
        scatter(k, rows, ssem)
        if k + 1 < BCH:
          nrows, ngsem, nssem = bufs[(k + 1) % 2]
          if k >= 1:
            pltpu.make_async_copy(nrows, agg_sh.at[dbuf.at[k - 1]],
                                  nssem).wait()
          gather(k + 1, nrows, ngsem).start()
      pltpu.make_async_copy(rows0, agg_sh.at[dbuf.at[BCH - 2]], ssem0).wait()
      pltpu.make_async_copy(rows1, agg_sh.at[dbuf.at[BCH - 1]], ssem1).wait()
      load_group(lax.rem(g + 1, NBLK))
      gather(0, rows0, gsem0).start()
      return carry

    lax.fori_loop(0, NBLK, grp, 0)
    gather(0, rows0, gsem0).wait()
    plsc.subcore_barrier()

    pltpu.sync_copy(agg_sh.at[pl.ds(s * RPT, RPT)], out_hbm.at[pl.ds(s * RPT, RPT)])

  @pl.when(c == 0)
  def _():
    pipe(xL, outL)

  @pl.when(c == 1)
  def _():
    pipe(xR, outR)


def _make_segmean():
  mesh = plsc.VectorSubcoreMesh(core_axis_name="c", subcore_axis_name="s")
  out_type = [jax.ShapeDtypeStruct((NP, HD), jnp.float32),
              jax.ShapeDtypeStruct((NP, HD), jnp.float32)]
  scratch = [
      pltpu.VMEM((BCH, CH), jnp.int32),
      pltpu.VMEM((BCH, CH), jnp.int32),
      pltpu.VMEM((BCH, CH), jnp.int32),
      pltpu.VMEM((CH, HD), jnp.float32),
      pltpu.VMEM((CH, HD), jnp.float32),
      pltpu.VMEM_SHARED((NP, HD), jnp.float32),
      pltpu.SemaphoreType.DMA,
      pltpu.SemaphoreType.DMA,
      pltpu.SemaphoreType.DMA,
      pltpu.SemaphoreType.DMA,
  ]
  return pl.kernel(_segmean_body, out_type=out_type, mesh=mesh,
                   scratch_types=scratch)


def _degree_body(dstp, zer, ones_h,
                 out0, out1,
                 dbuf, ones_v, deg_sh, sem):
  c = lax.axis_index("c")
  s = lax.axis_index("s")

  def zinit(i, carry):
    pltpu.sync_copy(zer, deg_sh.at[pl.ds(s * RPT + i * ZR, ZR)])
    return carry
  lax.fori_loop(0, RPT // ZR, zinit, 0)
  pltpu.sync_copy(ones_h, ones_v)
  plsc.subcore_barrier()

  def grp(b, carry):
    pltpu.sync_copy(dstp.at[s, c * (NBLK // 2) + b], dbuf)
    adds = [pltpu.async_copy(ones_v, deg_sh.at[dbuf.at[k]], sem, add=True)
            for k in range(BCH)]
    for a in adds:
      a.wait()
    return carry

  lax.fori_loop(0, NBLK // 2, grp, 0)
  plsc.subcore_barrier()

  @pl.when(c == 0)
  def _():
    pltpu.sync_copy(deg_sh.at[pl.ds(s * RPT, RPT)], out0.at[pl.ds(s * RPT, RPT)])

  @pl.when(c == 1)
  def _():
    pltpu.sync_copy(deg_sh.at[pl.ds(s * RPT, RPT)], out1.at[pl.ds(s * RPT, RPT)])


def _make_degree():
  mesh = plsc.VectorSubcoreMesh(core_axis_name="c", subcore_axis_name="s")
  out_type = [jax.ShapeDtypeStruct((NP, HD), jnp.float32),
              jax.ShapeDtypeStruct((NP, HD), jnp.float32)]
  scratch = [
      pltpu.VMEM((BCH, CH), jnp.int32),
      pltpu.VMEM((CH, HD), jnp.float32),
      pltpu.VMEM_SHARED((NP, HD), jnp.float32),
      pltpu.SemaphoreType.DMA,
  ]
  return pl.kernel(_degree_body, out_type=out_type, mesh=mesh,
                   scratch_types=scratch)


def _layer_body(relu, aL_ref, aR_ref, d0_ref, d1_ref, xL_ref, xR_ref,
                wl_ref, bl_ref, wr_ref, oL_ref, oR_ref):
  s = 1.0 / jnp.maximum(d0_ref[:, 0:1] + d1_ref[:, 0:1], 1.0)
  aL = aL_ref[...] * s
  aR = aR_ref[...] * s
  wl = wl_ref[...]
  wr = wr_ref[...]
  o = (jnp.dot(aL, wl[:HD], preferred_element_type=jnp.float32)
       + jnp.dot(aR, wl[HD:], preferred_element_type=jnp.float32)
       + jnp.dot(xL_ref[...], wr[:HD], preferred_element_type=jnp.float32)
       + jnp.dot(xR_ref[...], wr[HD:], preferred_element_type=jnp.float32)
       + bl_ref[...][None, :])
  if relu:
    o = jnp.maximum(o, 0.0)
  oL_ref[...] = o[:, :HD]
  oR_ref[...] = o[:, HD:]


def _tc_layer(aL, aR, d0, d1, xL, xR, Wl, bl, Wr, relu):
  grid = (NP // RBLK,)
  half = pl.BlockSpec((RBLK, HD), lambda i: (i, 0))
  return pl.pallas_call(
      functools.partial(_layer_body, relu),
      grid=grid,
      in_specs=[half, half, half, half, half, half,
                pl.BlockSpec((D, D), lambda i: (0, 0)),
                pl.BlockSpec((D,), lambda i: (0,)),
                pl.BlockSpec((D, D), lambda i: (0, 0))],
      out_specs=[half, half],
      out_shape=[jax.ShapeDtypeStruct((NP, HD), jnp.float32),
                 jax.ShapeDtypeStruct((NP, HD), jnp.float32)],
  )(aL, aR, d0, d1, xL, xR, Wl, bl, Wr)


def _head_body(aL_ref, aR_ref, d0_ref, d1_ref, xL_ref, xR_ref, wl_ref, bl_ref,
               wr_ref, wp1_ref, bp1_ref, wp2_ref, bp2_ref, wn1_ref, bn1_ref,
               wn2_ref, bn2_ref, z_ref, nul_ref):
  s = 1.0 / jnp.maximum(d0_ref[:, 0:1] + d1_ref[:, 0:1], 1.0)
  aL = aL_ref[...] * s
  aR = aR_ref[...] * s
  wl = wl_ref[...]
  wr = wr_ref[...]
  h = (jnp.dot(aL, wl[:HD], preferred_element_type=jnp.float32)
       + jnp.dot(aR, wl[HD:], preferred_element_type=jnp.float32)
       + jnp.dot(xL_ref[...], wr[:HD], preferred_element_type=jnp.float32)
       + jnp.dot(xR_ref[...], wr[HD:], preferred_element_type=jnp.float32)
       + bl_ref[...][None, :])
  t = jnp.maximum(jnp.dot(h, wp1_ref[...], preferred_element_type=jnp.float32)
                  + bp1_ref[...][None, :], 0.0)
  z0 = jnp.dot(t, wp2_ref[...], preferred_element_type=jnp.float32) + bp2_ref[...][None, :]
  nrm = jnp.sqrt(jnp.sum(z0 * z0, axis=1, keepdims=True))
  z = z0 / jnp.maximum(nrm, 1e-12)
  wn1 = wn1_ref[...]
  y = jnp.maximum(jnp.dot(h, wn1[:D], preferred_element_type=jnp.float32)
                  + jnp.dot(z, wn1[D:], preferred_element_type=jnp.float32)
                  + bn1_ref[...][None, :], 0.0)
  nul = jnp.sum(y * wn2_ref[...][None, :], axis=1) + bn2_ref[...]
  z_ref[...] = z
  nul_ref[...] = nul


def _tc_layer2_heads(aL, aR, d0, d1, xL, xR, Wl, bl, Wr,
                     Wp1, bp1, Wp2, bp2, Wn1, bn1, wn2, bn2):
  grid = (NP // RBLK,)
  half = pl.BlockSpec((RBLK, HD), lambda i: (i, 0))
  full = lambda shape: pl.BlockSpec(shape, (lambda i: (0,) * len(shape)))
  return pl.pallas_call(
      _head_body,
      grid=grid,
      in_specs=[half, half, half, half, half, half,
                full((D, D)), full((D,)), full((D, D)),
                full((D, HD)), full((HD,)), full((HD, HD)), full((HD,)),
                full((D + HD, 64)), full((64,)), full((64,)), full((1,))],
      out_specs=[pl.BlockSpec((RBLK, HD), lambda i: (i, 0)),
                 pl.BlockSpec((RBLK,), lambda i: (i,))],
      out_shape=[jax.ShapeDtypeStruct((NP, HD), jnp.float32),
                 jax.ShapeDtypeStruct((NP,), jnp.float32)],
  )(aL, aR, d0, d1, xL, xR, Wl, bl, Wr, Wp1, bp1, Wp2, bp2, Wn1, bn1, wn2, bn2)


def kernel(A_x, B_x, A_edge_index, B_edge_index,
           Wl0, bl0, Wr0, Wl1, bl1, Wr1, Wl2, bl2, Wr2,
           Wp1, bp1, Wp2, bp2, Wn1, bn1, Wn2, bn2):
  segmean = _make_segmean()
  degree = _make_degree()

  zeros_h = jnp.zeros((ZR, HD), jnp.float32)
  ones_h = jnp.ones((CH, HD), jnp.float32)
  wn2 = Wn2[:, 0]

  pad = E_PAD - E

  def prep_edges(ei):
    src = ei[0].astype(jnp.int32)
    dst = ei[1].astype(jnp.int32)
    srcp = jnp.concatenate([src, jnp.zeros((pad,), jnp.int32)])
    dpad = N + (jnp.arange(pad, dtype=jnp.int32) % (NP - N))
    dstp = jnp.concatenate([dst, dpad])

    def pack(a):
      a2 = a.reshape(NSUB, NBLK, BCH, CH // 2, 2)
      w = a2[..., 0] | (a2[..., 1] << 16)
      return w.reshape(NSUB, NBLK, BCH // 2, CH)

    idxp = jnp.concatenate([pack(srcp), pack(dstp)], axis=2)
    return idxp, dstp.reshape(NSUB, NBLK, BCH, CH)

  def prep_x(x):
    xp = jnp.zeros((NP, D), jnp.float32).at[:N].set(x)
    return xp[:, :HD], xp[:, HD:]

  def branch(x, ei):
    idxp, dstp = prep_edges(ei)
    xL, xR = prep_x(x)
    d0, d1 = degree(dstp, zeros_h, ones_h)
    aL0, aR0 = segmean(xL, xR, idxp, zeros_h)
    h1L, h1R = _tc_layer(aL0, aR0, d0, d1, xL, xR, Wl0, bl0, Wr0, True)
    aL1, aR1 = segmean(h1L, h1R, idxp, zeros_h)
    h2L, h2R = _tc_layer(aL1, aR1, d0, d1, h1L, h1R, Wl1, bl1, Wr1, True)
    aL2, aR2 = segmean(h2L, h2R, idxp, zeros_h)
    z, nul = _tc_layer2_heads(aL2, aR2, d0, d1, h2L, h2R, Wl2, bl2, Wr2,
                              Wp1, bp1, Wp2, bp2, Wn1, bn1, wn2, bn2)
    return z[:N], nul[:N]

  zA, nulA = branch(A_x, A_edge_index)
  zB, nulB = branch(B_x, B_edge_index)
  return (zA, zB, nulA, nulB)


# speedup vs baseline: 1.0348x; 1.0305x over previous
import functools

import jax
import jax.numpy as jnp
from jax import lax
from jax.experimental import pallas as pl
from jax.experimental.pallas import tpu as pltpu
from jax.experimental.pallas import tpu_sc as plsc

N = 10000
NP = 10240
E = 160000
E_PAD = 163840
D = 256
HD = 128
NSUB = 16
CH = 128
NBLK = 10
BCH = 8
NCH = NBLK * BCH
RPT = NP // NSUB
RBLK = 512


def _segmean_body(xL, xR, srcp, dstp, zer,
                  outL, outR,
                  sbuf, dbuf, rows, agg_sh, sem):
  c = lax.axis_index("c")
  s = lax.axis_index("s")

  def pipe(x_hbm, out_hbm):
    pltpu.sync_copy(zer, agg_sh.at[pl.ds(s * RPT, RPT)])
    plsc.subcore_barrier()

    def blk(b, carry):
      pltpu.sync_copy(srcp.at[s, b], sbuf)
      pltpu.sync_copy(dstp.at[s, b], dbuf)

      def it(k, carry2):
        gat = pltpu.make_async_copy(x_hbm.at[sbuf.at[k]], rows, sem)
        gat.start()
        gat.wait()
        pltpu.sync_copy(rows, agg_sh.at[dbuf.at[k]], add=True)
        return carry2

      return lax.fori_loop(0, BCH, it, carry)

    lax.fori_loop(0, NBLK, blk, 0)
    plsc.subcore_barrier()

    pltpu.sync_copy(agg_sh.at[pl.ds(s * RPT, RPT)], out_hbm.at[pl.ds(s * RPT, RPT)])

  @pl.when(c == 0)
  def _():
    pipe(xL, outL)

  @pl.when(c == 1)
  def _():
    pipe(xR, outR)


def _make_segmean():
  mesh = plsc.VectorSubcoreMesh(core_axis_name="c", subcore_axis_name="s")
  out_type = [jax.ShapeDtypeStruct((NP, HD), jnp.float32),
              jax.ShapeDtypeStruct((NP, HD), jnp.float32)]
  scratch = [
      pltpu.VMEM((BCH, CH), jnp.int32),
      pltpu.VMEM((BCH, CH), jnp.int32),
      pltpu.VMEM((CH, HD), jnp.float32),
      pltpu.VMEM_SHARED((NP, HD), jnp.float32),
      pltpu.SemaphoreType.DMA,
  ]
  return pl.kernel(_segmean_body, out_type=out_type, mesh=mesh,
                   scratch_types=scratch)


def _degree_body(dstp, zer, ones_h,
                 out0, out1,
                 dbuf, ones_v, deg_sh, sem):
  c = lax.axis_index("c")
  s = lax.axis_index("s")

  pltpu.sync_copy(zer, deg_sh.at[pl.ds(s * RPT, RPT)])
  pltpu.sync_copy(ones_h, ones_v)
  plsc.subcore_barrier()

  def blk(b, carry):
    pltpu.sync_copy(dstp.at[s, c * (NBLK // 2) + b], dbuf)
    adds = [pltpu.async_copy(ones_v, deg_sh.at[dbuf.at[k]], sem, add=True)
            for k in range(BCH)]
    for a in adds:
      a.wait()
    return carry

  lax.fori_loop(0, NBLK // 2, blk, 0)
  plsc.subcore_barrier()

  @pl.when(c == 0)
  def _():
    pltpu.sync_copy(deg_sh.at[pl.ds(s * RPT, RPT)], out0.at[pl.ds(s * RPT, RPT)])

  @pl.when(c == 1)
  def _():
    pltpu.sync_copy(deg_sh.at[pl.ds(s * RPT, RPT)], out1.at[pl.ds(s * RPT, RPT)])


def _make_degree():
  mesh = plsc.VectorSubcoreMesh(core_axis_name="c", subcore_axis_name="s")
  out_type = [jax.ShapeDtypeStruct((NP, HD), jnp.float32),
              jax.ShapeDtypeStruct((NP, HD), jnp.float32)]
  scratch = [
      pltpu.VMEM((BCH, CH), jnp.int32),
      pltpu.VMEM((CH, HD), jnp.float32),
      pltpu.VMEM_SHARED((NP, HD), jnp.float32),
      pltpu.SemaphoreType.DMA,
  ]
  return pl.kernel(_degree_body, out_type=out_type, mesh=mesh,
                   scratch_types=scratch)


def _layer_body(relu, aL_ref, aR_ref, d0_ref, d1_ref, xL_ref, xR_ref,
                wl_ref, bl_ref, wr_ref, oL_ref, oR_ref):
  s = 1.0 / jnp.maximum(d0_ref[:, 0:1] + d1_ref[:, 0:1], 1.0)
  aL = aL_ref[...] * s
  aR = aR_ref[...] * s
  wl = wl_ref[...]
  wr = wr_ref[...]
  o = (jnp.dot(aL, wl[:HD], preferred_element_type=jnp.float32)
       + jnp.dot(aR, wl[HD:], preferred_element_type=jnp.float32)
       + jnp.dot(xL_ref[...], wr[:HD], preferred_element_type=jnp.float32)
       + jnp.dot(xR_ref[...], wr[HD:], preferred_element_type=jnp.float32)
       + bl_ref[...][None, :])
  if relu:
    o = jnp.maximum(o, 0.0)
  oL_ref[...] = o[:, :HD]
  oR_ref[...] = o[:, HD:]


def _tc_layer(aL, aR, d0, d1, xL, xR, Wl, bl, Wr, relu):
  grid = (NP // RBLK,)
  half = pl.BlockSpec((RBLK, HD), lambda i: (i, 0))
  return pl.pallas_call(
      functools.partial(_layer_body, relu),
      grid=grid,
      in_specs=[half, half, half, half, half, half,
                pl.BlockSpec((D, D), lambda i: (0, 0)),
                pl.BlockSpec((D,), lambda i: (0,)),
                pl.BlockSpec((D, D), lambda i: (0, 0))],
      out_specs=[half, half],
      out_shape=[jax.ShapeDtypeStruct((NP, HD), jnp.float32),
                 jax.ShapeDtypeStruct((NP, HD), jnp.float32)],
  )(aL, aR, d0, d1, xL, xR, Wl, bl, Wr)


def _head_body(aL_ref, aR_ref, d0_ref, d1_ref, xL_ref, xR_ref, wl_ref, bl_ref,
               wr_ref, wp1_ref, bp1_ref, wp2_ref, bp2_ref, wn1_ref, bn1_ref,
               wn2_ref, bn2_ref, z_ref, nul_ref):
  s = 1.0 / jnp.maximum(d0_ref[:, 0:1] + d1_ref[:, 0:1], 1.0)
  aL = aL_ref[...] * s
  aR = aR_ref[...] * s
  wl = wl_ref[...]
  wr = wr_ref[...]
  h = (jnp.dot(aL, wl[:HD], preferred_element_type=jnp.float32)
       + jnp.dot(aR, wl[HD:], preferred_element_type=jnp.float32)
       + jnp.dot(xL_ref[...], wr[:HD], preferred_element_type=jnp.float32)
       + jnp.dot(xR_ref[...], wr[HD:], preferred_element_type=jnp.float32)
       + bl_ref[...][None, :])
  t = jnp.maximum(jnp.dot(h, wp1_ref[...], preferred_element_type=jnp.float32)
                  + bp1_ref[...][None, :], 0.0)
  z0 = jnp.dot(t, wp2_ref[...], preferred_element_type=jnp.float32) + bp2_ref[...][None, :]
  nrm = jnp.sqrt(jnp.sum(z0 * z0, axis=1, keepdims=True))
  z = z0 / jnp.maximum(nrm, 1e-12)
  wn1 = wn1_ref[...]
  y = jnp.maximum(jnp.dot(h, wn1[:D], preferred_element_type=jnp.float32)
                  + jnp.dot(z, wn1[D:], preferred_element_type=jnp.float32)
                  + bn1_ref[...][None, :], 0.0)
  nul = jnp.sum(y * wn2_ref[...][None, :], axis=1) + bn2_ref[...]
  z_ref[...] = z
  nul_ref[...] = nul


def _tc_layer2_heads(aL, aR, d0, d1, xL, xR, Wl, bl, Wr,
                     Wp1, bp1, Wp2, bp2, Wn1, bn1, wn2, bn2):
  grid = (NP // RBLK,)
  half = pl.BlockSpec((RBLK, HD), lambda i: (i, 0))
  full = lambda shape: pl.BlockSpec(shape, (lambda i: (0,) * len(shape)))
  return pl.pallas_call(
      _head_body,
      grid=grid,
      in_specs=[half, half, half, half, half, half,
                full((D, D)), full((D,)), full((D, D)),
                full((D, HD)), full((HD,)), full((HD, HD)), full((HD,)),
                full((D + HD, 64)), full((64,)), full((64,)), full((1,))],
      out_specs=[pl.BlockSpec((RBLK, HD), lambda i: (i, 0)),
                 pl.BlockSpec((RBLK,), lambda i: (i,))],
      out_shape=[jax.ShapeDtypeStruct((NP, HD), jnp.float32),
                 jax.ShapeDtypeStruct((NP,), jnp.float32)],
  )(aL, aR, d0, d1, xL, xR, Wl, bl, Wr, Wp1, bp1, Wp2, bp2, Wn1, bn1, wn2, bn2)


def kernel(A_x, B_x, A_edge_index, B_edge_index,
           Wl0, bl0, Wr0, Wl1, bl1, Wr1, Wl2, bl2, Wr2,
           Wp1, bp1, Wp2, bp2, Wn1, bn1, Wn2, bn2):
  segmean = _make_segmean()
  degree = _make_degree()

  zeros_h = jnp.zeros((RPT, HD), jnp.float32)
  ones_h = jnp.ones((CH, HD), jnp.float32)
  wn2 = Wn2[:, 0]

  pad = E_PAD - E

  def prep_edges(ei):
    src = ei[0].astype(jnp.int32)
    dst = ei[1].astype(jnp.int32)
    srcp = jnp.concatenate([src, jnp.zeros((pad,), jnp.int32)])
    dpad = N + (jnp.arange(pad, dtype=jnp.int32) % (NP - N))
    dstp = jnp.concatenate([dst, dpad])
    return (srcp.reshape(NSUB, NBLK, BCH, CH),
            dstp.reshape(NSUB, NBLK, BCH, CH))

  def prep_x(x):
    xp = jnp.zeros((NP, D), jnp.float32).at[:N].set(x)
    return xp[:, :HD], xp[:, HD:]

  def branch(x, ei):
    srcp, dstp = prep_edges(ei)
    xL, xR = prep_x(x)
    d0, d1 = degree(dstp, zeros_h, ones_h)
    aL0, aR0 = segmean(xL, xR, srcp, dstp, zeros_h)
    h1L, h1R = _tc_layer(aL0, aR0, d0, d1, xL, xR, Wl0, bl0, Wr0, True)
    aL1, aR1 = segmean(h1L, h1R, srcp, dstp, zeros_h)
    h2L, h2R = _tc_layer(aL1, aR1, d0, d1, h1L, h1R, Wl1, bl1, Wr1, True)
    aL2, aR2 = segmean(h2L, h2R, srcp, dstp, zeros_h)
    z, nul = _tc_layer2_heads(aL2, aR2, d0, d1, h2L, h2R, Wl2, bl2, Wr2,
                              Wp1, bp1, Wp2, bp2, Wn1, bn1, wn2, bn2)
    return z[:N], nul[:N]

  zA, nulA = branch(A_x, A_edge_index)
  zB, nulB = branch(B_x, B_edge_index)
  return (zA, zB, nulA, nulB)
